# baseline (speedup 1.0000x reference)
import functools

import jax
import jax.numpy as jnp
from jax import lax
from jax.experimental import pallas as pl
from jax.experimental.pallas import tpu as pltpu
from jax.experimental.pallas import tpu_sc as plsc

_K = 20
_INF = 3.0e38
_BIGI = 2 ** 31 - 1
_NC = 2
_NS = 16
_CAP = 64
_VPG = 8


def _dot(a, b, dims):
    return jax.lax.dot_general(a, b, (dims, ((), ())),
                               preferred_element_type=jnp.float32)


def _proj_kernel(x_ref, w1_ref, w2_ref, tt_ref, ta_ref, bs_ref):
    xb = x_ref[0]
    w1 = w1_ref[0]
    w2 = w2_ref[0]
    sc = w1.shape[0]
    tt = _dot(w1, xb, ((1,), (0,)))
    t = _dot(xb, w1, ((0,), (1,)))
    tt_ref[0] = tt
    w2n = w2[:, :sc]
    w2s = w2[:, sc:]
    a = _dot(tt, w2n, ((0,), (1,)))
    ta_ref[0] = jnp.concatenate([t, a], axis=1)
    bs_ref[0] = _dot(tt, w2s, ((0,), (1,)))


def _score_kernel(tt_ref, ta_ref, score_ref, *, rt):
    r = pl.program_id(1)
    tt = tt_ref[0]
    sc = tt.shape[0]
    sq_all = jnp.sum(tt * tt, axis=0, keepdims=True)
    rows_t = ta_ref[0, pl.ds(r * rt, rt), :sc]
    inner = _dot(rows_t, tt, ((1,), (0,)))
    score_ref[0] = sq_all - 2.0 * inner


def _finish_kernel(y_ref, out_ref):
    y = y_ref[0]
    sc = y.shape[1]
    ii = jax.lax.broadcasted_iota(jnp.int32, (sc, sc), 0)
    jj = jax.lax.broadcasted_iota(jnp.int32, (sc, sc), 1)
    eye = jnp.where(ii == jj, 1.0, 0.0).astype(jnp.float32)
    yt = _dot(eye, y, ((1,), (1,)))
    out_ref[0] = jnp.where(yt > 0, yt, jnp.exp(yt) - 1.0)


def _lmin(v, perms):
    for p in perms:
        v = jnp.minimum(v, jnp.take(v, p))
    return v


def _lsum(v, perms):
    for p in perms:
        v = v + jnp.take(v, p)
    return v


def _sc_attn(score_f, ta_f, bs_f, *, n, g_total, k, rb, rpw):
    ng = n // (16 * _VPG)
    nblk = rpw // rb
    f32 = jnp.float32
    i32 = jnp.int32

    def body(score_hbm, ta_hbm, bs_hbm, out_hbm,
             sb0, sb1, gmins, cv, ci, rowsb, bsb, outb, offr,
             sem0, sem1, semg):
        c = lax.axis_index("c")
        s = lax.axis_index("s")
        w = s * _NC + c
        row_lo = w * rpw
        gbase = (row_lo // n) * n
        iota = lax.broadcasted_iota(i32, (16,), 0)
        perms = tuple((iota + d) % 16 for d in (8, 4, 2, 1))
        inf16 = jnp.full((16,), _INF, f32)

        def start_blk(blk, sb, sem):
            src = score_hbm.at[pl.ds((row_lo + blk * rb) * n, rb * n)]
            pltpu.make_async_copy(src, sb, sem).start()

        def wait_blk(blk, sb, sem):
            src = score_hbm.at[pl.ds((row_lo + blk * rb) * n, rb * n)]
            pltpu.make_async_copy(src, sb, sem).wait()

        start_blk(0, sb0, sem0)
        start_blk(1, sb1, sem1)

        def topk_row(r, sb):
            base = r * n

            def p1(gi, ab):
                a1, a2 = ab
                gm = inf16
                for p in range(_VPG):
                    v = sb[pl.ds(base + (gi * _VPG + p) * 16, 16)]
                    t1 = jnp.minimum(a1, v)
                    a2 = jnp.minimum(a2, jnp.maximum(a1, v))
                    a1 = t1
                    gm = jnp.minimum(gm, v)
                gmins[pl.ds(gi * 16, 16)] = gm
                return (a1, a2)

            a1, a2 = lax.fori_loop(0, ng, p1, (inf16, inf16))

            def tau_round(_, car):
                b1, b2, _t = car
                m = _lmin(jnp.minimum(b1, b2), perms)
                b1 = jnp.where(b1 == m, _INF, b1)
                b2 = jnp.where(b2 == m, _INF, b2)
                return (b1, b2, m)

            _, _, tau = lax.fori_loop(0, k, tau_round, (a1, a2, inf16))
            tau_s = tau[0]

            for j in range(_CAP // 16):
                cv[pl.ds(j * 16, 16)] = inf16
            offr[0] = 0

            def extract_hit(valv, lanev, jbase):
                hitv = valv <= tau
                o = jnp.minimum(offr[0], _CAP - 1)
                blk16 = (o // 16) * 16
                sel = hitv & (iota == (o - blk16))
                cv[pl.ds(blk16, 16)] = jnp.where(
                    sel, valv, cv[pl.ds(blk16, 16)])
                jv = jnp.full((16,), jbase, i32) + lanev
                ci[pl.ds(blk16, 16)] = jnp.where(
                    sel, jv, ci[pl.ds(blk16, 16)])
                offr[0] = o + jnp.where(hitv, 1, 0)[0]

            def grp_body(gi, _):
                gm = gmins[pl.ds(gi * 16, 16)]

                @pl.when(_lmin(gm, perms)[0] <= tau_s)
                def _():
                    for p in range(_VPG):
                        v = sb[pl.ds(base + (gi * _VPG + p) * 16, 16)]
                        hv = v <= tau
                        lane0 = _lmin(jnp.where(hv, iota, 16), perms)

                        @pl.when(lane0[0] < 16)
                        def _(v=v, hv=hv, lane0=lane0, p=p):
                            jb = (gi * _VPG + p) * 16 + gbase
                            lanev = lane0
                            hh = hv
                            for _pass in range(4):
                                valv = _lmin(
                                    jnp.where(iota == lanev, v, _INF), perms)
                                extract_hit(valv, lanev, jb)
                                hh = hh & (iota != lanev)
                                lanev = _lmin(jnp.where(hh, iota, 16), perms)

                return 0

            lax.fori_loop(0, ng, grp_body, 0)

            vs = [cv[pl.ds(j * 16, 16)] for j in range(_CAP // 16)]
            cs = [ci[pl.ds(j * 16, 16)] for j in range(_CAP // 16)]
            for kk in range(k):
                u = vs[0]
                for j in range(1, len(vs)):
                    u = jnp.minimum(u, vs[j])
                m = _lmin(u, perms)
                hs = [v == m for v in vs]
                im = jnp.full((16,), _BIGI, i32)
                for j in range(len(vs)):
                    im = jnp.minimum(im, jnp.where(hs[j], cs[j], _BIGI))
                idx = _lmin(im, perms)[0]
                vs = [jnp.where(h, _INF, v) for h, v in zip(hs, vs)]
                pltpu.async_copy(
                    ta_hbm.at[pl.ds(idx * 32, 32)],
                    rowsb.at[pl.ds((r * k + kk) * 32, 32)], semg)
            return 0

        def attn_row(r, _):
            bvec = bsb[pl.ds(r * 16, 16)]
            num = jnp.zeros((16,), f32)
            den = jnp.zeros((16,), f32)
            for kk in range(k):
                o = (r * k + kk) * 32
                tv = rowsb[pl.ds(o, 16)]
                av = rowsb[pl.ds(o + 16, 16)]
                e = av + bvec
                e = jnp.where(e >= 0.0, e, 0.2 * e)
                wgt = jnp.exp(e)
                den = den + wgt
                num = num + wgt * tv
            outb[pl.ds(r * 16, 16)] = num / den
            return 0

        def do_block(blk, sb, sem):
            wait_blk(blk, sb, sem)
            pltpu.sync_copy(
                bs_hbm.at[pl.ds((row_lo + blk * rb) * 16, rb * 16)], bsb)

            def row_body(r, _):
                topk_row(r, sb)
                return 0

            lax.fori_loop(0, rb, row_body, 0)
            pltpu.make_async_copy(
                ta_hbm.at[pl.ds(0, rb * k * 32)], rowsb, semg).wait()
            lax.fori_loop(0, rb, attn_row, 0)
            pltpu.sync_copy(
                outb, out_hbm.at[pl.ds((row_lo + blk * rb) * 16, rb * 16)])

            @pl.when(blk + 2 < nblk)
            def _():
                start_blk(blk + 2, sb, sem)

        def pair_body(p, _):
            do_block(2 * p, sb0, sem0)
            do_block(2 * p + 1, sb1, sem1)
            return 0

        lax.fori_loop(0, nblk // 2, pair_body, 0)

    mesh = plsc.VectorSubcoreMesh(core_axis_name="c", subcore_axis_name="s",
                                  num_cores=_NC, num_subcores=_NS)
    return pl.kernel(
        body,
        out_type=jax.ShapeDtypeStruct((g_total * n * 16,), f32),
        mesh=mesh,
        scratch_types=[
            pltpu.VMEM((rb * n,), f32),
            pltpu.VMEM((rb * n,), f32),
            pltpu.VMEM((ng * 16,), f32),
            pltpu.VMEM((_CAP,), f32),
            pltpu.VMEM((_CAP,), i32),
            pltpu.VMEM((rb * k * 32,), f32),
            pltpu.VMEM((rb * 16,), f32),
            pltpu.VMEM((rb * 16,), f32),
            pltpu.SMEM((8,), i32),
            pltpu.SemaphoreType.DMA,
            pltpu.SemaphoreType.DMA,
            pltpu.SemaphoreType.DMA,
        ],
    )(score_f, ta_f, bs_f)


def _run(x, w1, w2, k):
    b, cin, n = x.shape
    h, sc, _ = w1.shape
    g = b * h
    rt = min(256, n)
    nt = n // rt
    f32 = jnp.float32
    rpw = (g * n) // (_NC * _NS)
    rb = min(8, rpw)

    tt, ta, bs = pl.pallas_call(
        _proj_kernel,
        grid=(g,),
        in_specs=[
            pl.BlockSpec((1, cin, n), lambda i: (i // h, 0, 0)),
            pl.BlockSpec((1, sc, cin), lambda i: (i % h, 0, 0)),
            pl.BlockSpec((1, sc, 2 * sc), lambda i: (i % h, 0, 0)),
        ],
        out_specs=[
            pl.BlockSpec((1, sc, n), lambda i: (i, 0, 0)),
            pl.BlockSpec((1, n, 2 * sc), lambda i: (i, 0, 0)),
            pl.BlockSpec((1, n, sc), lambda i: (i, 0, 0)),
        ],
        out_shape=[
            jax.ShapeDtypeStruct((g, sc, n), f32),
            jax.ShapeDtypeStruct((g, n, 2 * sc), f32),
            jax.ShapeDtypeStruct((g, n, sc), f32),
        ],
    )(x, w1, w2)

    score = pl.pallas_call(
        functools.partial(_score_kernel, rt=rt),
        grid=(g, nt),
        in_specs=[
            pl.BlockSpec((1, sc, n), lambda i, r: (i, 0, 0)),
            pl.BlockSpec((1, n, 2 * sc), lambda i, r: (i, 0, 0)),
        ],
        out_specs=pl.BlockSpec((1, rt, n), lambda i, r: (i, r, 0)),
        out_shape=jax.ShapeDtypeStruct((g, n, n), f32),
    )(tt, ta)

    y = _sc_attn(
        score.reshape(g * n * n),
        ta.reshape(g * n * 2 * sc),
        bs.reshape(g * n * sc),
        n=n, g_total=g, k=k, rb=rb, rpw=rpw,
    )

    out = pl.pallas_call(
        _finish_kernel,
        grid=(g,),
        in_specs=[pl.BlockSpec((1, n, sc), lambda i: (i, 0, 0))],
        out_specs=pl.BlockSpec((1, sc, n), lambda i: (i // h, i % h, 0)),
        out_shape=jax.ShapeDtypeStruct((b, h * sc, n), f32),
    )(y.reshape(g, n, sc))
    return out


@jax.jit
def kernel(x, W1, W2):
    return _run(x, W1, W2, _K)
